# baseline (speedup 1.0000x reference)
import functools

import jax
import jax.numpy as jnp
from jax.experimental import pallas as pl
from jax.experimental.pallas import tpu as pltpu

_WS = 8
_K = 9

_DN_WT = (((1,), (1,)), ((), ()))


def _linear(xrows, w_ref, b_ref, i):
    out = jax.lax.dot_general(xrows, w_ref[i], _DN_WT,
                              precision=None,
                              preferred_element_type=jnp.float32)
    return out + b_ref[i].reshape(1, -1)


def _fused_kernel(x_ref, wfc1_ref, bfc1_ref, wg_ref, bg_ref, wfc2_ref,
                  bfc2_ref, wf1_ref, bf1_ref, wf2_ref, bf2_ref, out_ref,
                  *, nb):
    _, C, ws, Wf = x_ref.shape
    Gw = Wf // ws
    N = ws * ws
    R = Gw * N
    hi = jax.lax.Precision.HIGHEST
    X = (x_ref[0].reshape(C, ws, Gw, ws)
         .transpose(2, 1, 3, 0)
         .reshape(R, C))
    for i in range(nb):
        h = _linear(X, wfc1_ref, bfc1_ref, i)
        hw = h.reshape(Gw, N, C)
        sq = jnp.sum(hw * hw, axis=2)
        dots = jax.lax.dot_general(
            hw, hw, (((2,), (2,)), ((0,), (0,))),
            precision=None, preferred_element_type=jnp.float32)
        d = sq[:, :, None] - 2.0 * dots + sq[:, None, :]
        jvals = jax.lax.broadcasted_iota(jnp.int32, (Gw, N, N), 2)
        ivals = jax.lax.broadcasted_iota(jnp.int32, (Gw, N, N), 1)
        relmax = hw
        dcur = jnp.where(jvals == ivals, jnp.float32(jnp.inf), d)
        for _ in range(_K - 1):
            amin = jnp.argmin(dcur, axis=2)
            first = jvals == amin[:, :, None]
            sel = first.astype(jnp.float32)
            gathered = jax.lax.dot_general(
                sel, hw, (((2,), (1,)), ((0,), (0,))),
                precision=hi, preferred_element_type=jnp.float32)
            relmax = jnp.maximum(relmax, gathered)
            dcur = jnp.where(first, jnp.float32(jnp.inf), dcur)
        rel = relmax - hw
        feat = jnp.concatenate([hw, rel], axis=2).reshape(R, 2 * C)
        g = jnp.maximum(_linear(feat, wg_ref, bg_ref, i), 0.0)
        X = _linear(g, wfc2_ref, bfc2_ref, i) + X
        y = jnp.maximum(_linear(X, wf1_ref, bf1_ref, i), 0.0)
        X = _linear(y, wf2_ref, bf2_ref, i) + X
    out_ref[0] = (X.reshape(Gw, ws, ws, C)
                  .transpose(3, 1, 0, 2)
                  .reshape(C, ws, Wf))


def kernel(x, Wfc1, bfc1, Wg, bg, Wfc2, bfc2, Wf1, bf1, Wf2, bf2):
    B, C, H, W = x.shape
    ws = _WS
    nh = H // ws
    nb = Wfc1.shape[0]

    full = lambda arr: pl.BlockSpec(arr.shape, lambda i: (0,) * arr.ndim)
    args = (x, Wfc1, bfc1, Wg, bg, Wfc2, bfc2, Wf1, bf1, Wf2, bf2)
    return pl.pallas_call(
        functools.partial(_fused_kernel, nb=nb),
        grid=(nh,),
        in_specs=[pl.BlockSpec((1, C, ws, W), lambda i: (0, 0, i, 0))]
                 + [full(a) for a in args[1:]],
        out_specs=pl.BlockSpec((1, C, ws, W), lambda i: (0, 0, i, 0)),
        out_shape=jax.ShapeDtypeStruct((B, C, H, W), jnp.float32),
        compiler_params=pltpu.CompilerParams(
            dimension_semantics=("parallel",)),
    )(*args)


# speedup vs baseline: 1.2866x; 1.2866x over previous
import functools

import jax
import jax.numpy as jnp
from jax.experimental import pallas as pl
from jax.experimental.pallas import tpu as pltpu

_WS = 8
_K = 9

_DN_WT = (((1,), (1,)), ((), ()))


def _linear(xrows, w_ref, b_ref, i):
    out = jax.lax.dot_general(xrows, w_ref[i], _DN_WT,
                              precision=None,
                              preferred_element_type=jnp.float32)
    return out + b_ref[i].reshape(1, -1)


def _fused_kernel(xw_ref, wfc1_ref, bfc1_ref, wg_ref, bg_ref, wfc2_ref,
                  bfc2_ref, wf1_ref, bf1_ref, wf2_ref, bf2_ref, out_ref,
                  *, nb):
    Gw, N, C = xw_ref.shape
    R = Gw * N
    hi = jax.lax.Precision.HIGHEST
    X = xw_ref[...].reshape(R, C)
    for i in range(nb):
        h = _linear(X, wfc1_ref, bfc1_ref, i)
        hw = h.reshape(Gw, N, C)
        sq = jnp.sum(hw * hw, axis=2)
        dots = jax.lax.dot_general(
            hw, hw, (((2,), (2,)), ((0,), (0,))),
            precision=None, preferred_element_type=jnp.float32)
        d = sq[:, :, None] - 2.0 * dots + sq[:, None, :]
        jvals = jax.lax.broadcasted_iota(jnp.int32, (Gw, N, N), 2)
        ivals = jax.lax.broadcasted_iota(jnp.int32, (Gw, N, N), 1)
        relmax = hw
        dcur = jnp.where(jvals == ivals, jnp.float32(jnp.inf), d)
        for _ in range(_K - 1):
            amin = jnp.argmin(dcur, axis=2)
            first = jvals == amin[:, :, None]
            sel = first.astype(jnp.float32)
            gathered = jax.lax.dot_general(
                sel, hw, (((2,), (1,)), ((0,), (0,))),
                precision=hi, preferred_element_type=jnp.float32)
            relmax = jnp.maximum(relmax, gathered)
            dcur = jnp.where(first, jnp.float32(jnp.inf), dcur)
        rel = relmax - hw
        feat = jnp.concatenate([hw, rel], axis=2).reshape(R, 2 * C)
        g = jnp.maximum(_linear(feat, wg_ref, bg_ref, i), 0.0)
        X = _linear(g, wfc2_ref, bfc2_ref, i) + X
        y = jnp.maximum(_linear(X, wf1_ref, bf1_ref, i), 0.0)
        X = _linear(y, wf2_ref, bf2_ref, i) + X
    out_ref[...] = X.reshape(Gw, N, C)


def kernel(x, Wfc1, bfc1, Wg, bg, Wfc2, bfc2, Wf1, bf1, Wf2, bf2):
    B, C, H, W = x.shape
    ws = _WS
    nh, nw = H // ws, W // ws
    NW = B * nh * nw
    N = ws * ws
    nb = Wfc1.shape[0]
    G = 16 if NW % 16 == 0 else 1

    xw = (x.reshape(B, C, nh, ws, nw, ws)
          .transpose(0, 2, 4, 3, 5, 1)
          .reshape(NW, N, C))

    full = lambda arr: pl.BlockSpec(arr.shape, lambda i: (0,) * arr.ndim)
    args = (xw, Wfc1, bfc1, Wg, bg, Wfc2, bfc2, Wf1, bf1, Wf2, bf2)
    out = pl.pallas_call(
        functools.partial(_fused_kernel, nb=nb),
        grid=(NW // G,),
        in_specs=[pl.BlockSpec((G, N, C), lambda i: (i, 0, 0))]
                 + [full(a) for a in args[1:]],
        out_specs=pl.BlockSpec((G, N, C), lambda i: (i, 0, 0)),
        out_shape=jax.ShapeDtypeStruct((NW, N, C), jnp.float32),
        compiler_params=pltpu.CompilerParams(
            dimension_semantics=("parallel",)),
    )(*args)

    return (out.reshape(B, nh, nw, ws, ws, C)
            .transpose(0, 5, 1, 3, 2, 4)
            .reshape(B, C, H, W))


# speedup vs baseline: 1.3350x; 1.0377x over previous
import functools

import jax
import jax.numpy as jnp
from jax.experimental import pallas as pl
from jax.experimental.pallas import tpu as pltpu

_WS = 8
_K = 9

_DN_WT = (((1,), (1,)), ((), ()))


def _linear(xrows, w_ref, b_ref, i):
    out = jax.lax.dot_general(xrows, w_ref[i], _DN_WT,
                              precision=None,
                              preferred_element_type=jnp.float32)
    return out + b_ref[i].reshape(1, -1)


def _fused_kernel(xw_ref, wfc1_ref, bfc1_ref, wg_ref, bg_ref, wfc2_ref,
                  bfc2_ref, wf1_ref, bf1_ref, wf2_ref, bf2_ref, out_ref,
                  *, nb):
    Gw, N, C = xw_ref.shape
    R = Gw * N
    hi = jax.lax.Precision.HIGHEST
    X = xw_ref[...].reshape(R, C)
    for i in range(nb):
        h = _linear(X, wfc1_ref, bfc1_ref, i)
        hw = h.reshape(Gw, N, C)
        sq = jnp.sum(hw * hw, axis=2)
        dots = jax.lax.dot_general(
            hw, hw, (((2,), (2,)), ((0,), (0,))),
            precision=None, preferred_element_type=jnp.float32)
        d = sq[:, :, None] - 2.0 * dots + sq[:, None, :]
        jvals = jax.lax.broadcasted_iota(jnp.int32, (Gw, N, N), 2)
        ivals = jax.lax.broadcasted_iota(jnp.int32, (Gw, N, N), 1)
        relmax = hw
        dcur = jnp.where(jvals == ivals, jnp.float32(jnp.inf), d)
        for _ in range(_K - 1):
            amin = jnp.argmin(dcur, axis=2)
            first = jvals == amin[:, :, None]
            sel = first.astype(jnp.float32)
            gathered = jax.lax.dot_general(
                sel, hw, (((2,), (1,)), ((0,), (0,))),
                precision=hi, preferred_element_type=jnp.float32)
            relmax = jnp.maximum(relmax, gathered)
            dcur = jnp.where(first, jnp.float32(jnp.inf), dcur)
        rel = relmax - hw
        feat = jnp.concatenate([hw, rel], axis=2).reshape(R, 2 * C)
        g = jnp.maximum(_linear(feat, wg_ref, bg_ref, i), 0.0)
        X = _linear(g, wfc2_ref, bfc2_ref, i) + X
        y = jnp.maximum(_linear(X, wf1_ref, bf1_ref, i), 0.0)
        X = _linear(y, wf2_ref, bf2_ref, i) + X
    out_ref[...] = X.reshape(Gw, N, C)


def kernel(x, Wfc1, bfc1, Wg, bg, Wfc2, bfc2, Wf1, bf1, Wf2, bf2):
    B, C, H, W = x.shape
    ws = _WS
    nh, nw = H // ws, W // ws
    NW = B * nh * nw
    N = ws * ws
    nb = Wfc1.shape[0]
    G = 28 if NW % 28 == 0 else 1

    xw = (x.reshape(B, C, nh, ws, nw, ws)
          .transpose(0, 2, 4, 3, 5, 1)
          .reshape(NW, N, C))

    full = lambda arr: pl.BlockSpec(arr.shape, lambda i: (0,) * arr.ndim)
    args = (xw, Wfc1, bfc1, Wg, bg, Wfc2, bfc2, Wf1, bf1, Wf2, bf2)
    out = pl.pallas_call(
        functools.partial(_fused_kernel, nb=nb),
        grid=(NW // G,),
        in_specs=[pl.BlockSpec((G, N, C), lambda i: (i, 0, 0))]
                 + [full(a) for a in args[1:]],
        out_specs=pl.BlockSpec((G, N, C), lambda i: (i, 0, 0)),
        out_shape=jax.ShapeDtypeStruct((NW, N, C), jnp.float32),
        compiler_params=pltpu.CompilerParams(
            dimension_semantics=("parallel",)),
    )(*args)

    return (out.reshape(B, nh, nw, ws, ws, C)
            .transpose(0, 5, 1, 3, 2, 4)
            .reshape(B, C, H, W))


# speedup vs baseline: 1.3605x; 1.0191x over previous
import functools

import jax
import jax.numpy as jnp
from jax.experimental import pallas as pl
from jax.experimental.pallas import tpu as pltpu

_WS = 8
_K = 9

_DN_WT = (((1,), (1,)), ((), ()))


def _linear(xrows, w_ref, b_ref, i):
    out = jax.lax.dot_general(xrows, w_ref[i], _DN_WT,
                              precision=None,
                              preferred_element_type=jnp.float32)
    return out + b_ref[i].reshape(1, -1)


def _fused_kernel(xw_ref, wfc1_ref, bfc1_ref, wg_ref, bg_ref, wfc2_ref,
                  bfc2_ref, wf1_ref, bf1_ref, wf2_ref, bf2_ref, out_ref,
                  *, nb):
    Gw, N, C = xw_ref.shape
    R = Gw * N
    hi = jax.lax.Precision.HIGHEST
    X = xw_ref[...].reshape(R, C)
    for i in range(nb):
        h = _linear(X, wfc1_ref, bfc1_ref, i)
        hw = h.reshape(Gw, N, C)
        sq = jnp.sum(hw * hw, axis=2)
        dots = jax.lax.dot_general(
            hw, hw, (((2,), (2,)), ((0,), (0,))),
            precision=None, preferred_element_type=jnp.float32)
        d = sq[:, :, None] - 2.0 * dots + sq[:, None, :]
        jvals = jax.lax.broadcasted_iota(jnp.int32, (Gw, N, N), 2)
        ivals = jax.lax.broadcasted_iota(jnp.int32, (Gw, N, N), 1)
        relmax = hw
        dcur = jnp.where(jvals == ivals, jnp.float32(jnp.inf), d)
        for _ in range(_K - 1):
            amin = jnp.argmin(dcur, axis=2)
            first = jvals == amin[:, :, None]
            sel = first.astype(jnp.float32)
            gathered = jax.lax.dot_general(
                sel, hw, (((2,), (1,)), ((0,), (0,))),
                precision=hi, preferred_element_type=jnp.float32)
            relmax = jnp.maximum(relmax, gathered)
            dcur = jnp.where(first, jnp.float32(jnp.inf), dcur)
        rel = relmax - hw
        feat = jnp.concatenate([hw, rel], axis=2).reshape(R, 2 * C)
        g = jnp.maximum(_linear(feat, wg_ref, bg_ref, i), 0.0)
        X = _linear(g, wfc2_ref, bfc2_ref, i) + X
        y = jnp.maximum(_linear(X, wf1_ref, bf1_ref, i), 0.0)
        X = _linear(y, wf2_ref, bf2_ref, i) + X
    out_ref[...] = X.reshape(Gw, N, C)


def kernel(x, Wfc1, bfc1, Wg, bg, Wfc2, bfc2, Wf1, bf1, Wf2, bf2):
    B, C, H, W = x.shape
    ws = _WS
    nh, nw = H // ws, W // ws
    NW = B * nh * nw
    N = ws * ws
    nb = Wfc1.shape[0]
    G = 49 if NW % 49 == 0 else 1

    xw = (x.reshape(B, C, nh, ws, nw, ws)
          .transpose(0, 2, 4, 3, 5, 1)
          .reshape(NW, N, C))

    full = lambda arr: pl.BlockSpec(arr.shape, lambda i: (0,) * arr.ndim)
    args = (xw, Wfc1, bfc1, Wg, bg, Wfc2, bfc2, Wf1, bf1, Wf2, bf2)
    out = pl.pallas_call(
        functools.partial(_fused_kernel, nb=nb),
        grid=(NW // G,),
        in_specs=[pl.BlockSpec((G, N, C), lambda i: (i, 0, 0))]
                 + [full(a) for a in args[1:]],
        out_specs=pl.BlockSpec((G, N, C), lambda i: (i, 0, 0)),
        out_shape=jax.ShapeDtypeStruct((NW, N, C), jnp.float32),
        compiler_params=pltpu.CompilerParams(
            dimension_semantics=("parallel",)),
    )(*args)

    return (out.reshape(B, nh, nw, ws, ws, C)
            .transpose(0, 5, 1, 3, 2, 4)
            .reshape(B, C, H, W))
